# baseline (speedup 1.0000x reference)
import functools

import jax
import jax.numpy as jnp
from jax import lax
from jax.experimental import pallas as pl
from jax.experimental.pallas import tpu as pltpu
from jax.experimental.pallas import tpu_sc as plsc

B, L = 4096, 200
EMB, HID, NCLS = 64, 128, 2
NW = 32
BAGS_PER_W = B // NW
HALF = L // 2
NLANE = 16

_mesh = plsc.VectorSubcoreMesh(core_axis_name="c", subcore_axis_name="s")


@functools.partial(
    pl.kernel,
    out_type=jax.ShapeDtypeStruct((B, EMB), jnp.float32),
    mesh=_mesh,
    scratch_types=[
        pltpu.VMEM((2 * BAGS_PER_W, HALF), jnp.int32),
        pltpu.VMEM((L, EMB), jnp.float32),
        pltpu.VMEM((L, EMB), jnp.float32),
        pltpu.VMEM((BAGS_PER_W, EMB), jnp.float32),
        pltpu.SemaphoreType.DMA,
        pltpu.SemaphoreType.DMA,
    ],
    compiler_params=pltpu.CompilerParams(use_tc_tiling_on_sc=False),
)
def _embbag_mean(idx_hbm, table_hbm, out_hbm, idx_v, rows0, rows1, outbuf, sem0, sem1):
    wid = lax.axis_index("s") * 2 + lax.axis_index("c")
    base = wid * BAGS_PER_W
    pltpu.sync_copy(idx_hbm.at[pl.ds(2 * base, 2 * BAGS_PER_W)], idx_v)

    def gather(bag, rows, sem):
        r = 2 * bag
        pltpu.async_copy(table_hbm.at[idx_v.at[r]], rows.at[pl.ds(0, HALF)], sem)
        pltpu.async_copy(table_hbm.at[idx_v.at[r + 1]], rows.at[pl.ds(HALF, HALF)], sem)

    def drain(rows, sem):
        pltpu.make_async_copy(table_hbm.at[idx_v.at[0]], rows.at[pl.ds(0, HALF)], sem).wait()
        pltpu.make_async_copy(table_hbm.at[idx_v.at[0]], rows.at[pl.ds(HALF, HALF)], sem).wait()

    def accumulate(rows):
        def body(i, accs):
            a0, a1, a2, a3 = accs
            for dr in range(4):
                r = i * 4 + dr
                a0 = a0 + rows[r, pl.ds(0, NLANE)]
                a1 = a1 + rows[r, pl.ds(NLANE, NLANE)]
                a2 = a2 + rows[r, pl.ds(2 * NLANE, NLANE)]
                a3 = a3 + rows[r, pl.ds(3 * NLANE, NLANE)]
            return a0, a1, a2, a3

        z = jnp.zeros((NLANE,), jnp.float32)
        return lax.fori_loop(0, L // 4, body, (z, z, z, z))

    scale = jnp.float32(1.0 / L)

    def store(bag, accs):
        a0, a1, a2, a3 = accs
        outbuf[bag, pl.ds(0, NLANE)] = a0 * scale
        outbuf[bag, pl.ds(NLANE, NLANE)] = a1 * scale
        outbuf[bag, pl.ds(2 * NLANE, NLANE)] = a2 * scale
        outbuf[bag, pl.ds(3 * NLANE, NLANE)] = a3 * scale

    gather(0, rows0, sem0)
    gather(1, rows1, sem1)

    def step(i, carry):
        bag = 2 * i
        drain(rows0, sem0)
        store(bag, accumulate(rows0))
        gather(jnp.minimum(bag + 2, BAGS_PER_W - 2), rows0, sem0)
        drain(rows1, sem1)
        store(bag + 1, accumulate(rows1))
        gather(jnp.minimum(bag + 3, BAGS_PER_W - 1), rows1, sem1)
        return carry

    lax.fori_loop(0, BAGS_PER_W // 2, step, 0)
    drain(rows0, sem0)
    drain(rows1, sem1)
    pltpu.sync_copy(outbuf, out_hbm.at[pl.ds(base, BAGS_PER_W)])


def _mlp_body(x_ref, w1_ref, b1_ref, w2_ref, b2_ref, o_ref):
    x = x_ref[...]
    h = jnp.dot(x, w1_ref[...], preferred_element_type=jnp.float32) + b1_ref[...]
    h = jnp.maximum(h, 0.0)
    logits = jnp.dot(h, w2_ref[...], preferred_element_type=jnp.float32) + b2_ref[...]
    m = jnp.max(logits, axis=1, keepdims=True)
    e = jnp.exp(logits - m)
    o_ref[...] = e / jnp.sum(e, axis=1, keepdims=True)


_mlp = pl.pallas_call(
    _mlp_body,
    out_shape=jax.ShapeDtypeStruct((B, NCLS), jnp.float32),
)


@jax.jit
def kernel(batch_inputs, batch_lengths, emb_table, W1, b1, W2, b2):
    del batch_lengths
    idx = batch_inputs.astype(jnp.int32).reshape(2 * B, HALF)
    pooled = _embbag_mean(idx, emb_table)
    return _mlp(pooled, W1.T, b1.reshape(1, HID), W2.T, b2.reshape(1, NCLS))
